# baseline (speedup 1.0000x reference)
import functools

import jax
import jax.numpy as jnp
from jax import lax
from jax.experimental import pallas as pl
from jax.experimental.pallas import tpu as pltpu
from jax.experimental.pallas import tpu_sc as plsc

EMB = 128
CHUNK = 400
IB = 16
LANES = 16


def _sc_embedding_lookup(x_flat, table_flat, n_rows):
    n = x_flat.shape[0]
    info = plsc.get_sparse_core_info()
    nw = info.num_cores * info.num_subcores
    per_w = n // nw
    n_iters = per_w // CHUNK
    assert per_w % CHUNK == 0 and n % nw == 0 and n_iters % IB == 0
    assert n_iters % 2 == 0

    mesh = plsc.VectorSubcoreMesh(core_axis_name="c", subcore_axis_name="s")

    @functools.partial(
        pl.kernel,
        mesh=mesh,
        compiler_params=pltpu.CompilerParams(needs_layout_passes=False),
        out_type=jax.ShapeDtypeStruct((n * EMB,), jnp.float32),
        scratch_types=[
            pltpu.VMEM((n_rows * EMB,), jnp.float32),
            pltpu.VMEM((IB * CHUNK,), jnp.int32),
            pltpu.VMEM((CHUNK * EMB,), jnp.float32),
            pltpu.VMEM((CHUNK * EMB,), jnp.float32),
            pltpu.SemaphoreType.DMA,
        ],
    )
    def k(x_hbm, table_hbm, out_hbm, table_v, idx_v, rows0, rows1, sem_s):
        wid = lax.axis_index("s") * info.num_cores + lax.axis_index("c")
        base = wid * per_w
        pltpu.sync_copy(table_hbm, table_v)
        bufs = (rows0, rows1)

        def expand_chunk(i, buf):
            j = i % IB

            def g_body(g, c2):
                idx16 = idx_v[pl.ds(j * CHUNK + g * LANES, LANES)]
                base_l = idx16 << 7
                base_s = (lax.iota(jnp.int32, LANES) + g * LANES) << 7
                for c in range(EMB):
                    v = plsc.load_gather(table_v, [base_l + c])
                    plsc.store_scatter(buf, [base_s + c], v)
                return c2

            lax.fori_loop(0, CHUNK // LANES, g_body, 0)

        def body(i2, carry):
            for s in range(2):
                i = i2 * 2 + s
                buf = bufs[s]
                start = base + i * CHUNK

                @pl.when(i % IB == 0)
                def _():
                    pltpu.sync_copy(x_hbm.at[pl.ds(start, IB * CHUNK)], idx_v)

                expand_chunk(i, buf)

                @pl.when(i >= 1)
                def _():
                    pltpu.make_async_copy(
                        bufs[1 - s],
                        out_hbm.at[pl.ds(base * EMB, CHUNK * EMB)],
                        sem_s,
                    ).wait()

                pltpu.async_copy(
                    buf, out_hbm.at[pl.ds(start * EMB, CHUNK * EMB)], sem_s
                )
            return carry

        lax.fori_loop(0, n_iters // 2, body, 0)
        pltpu.make_async_copy(
            bufs[1], out_hbm.at[pl.ds(base * EMB, CHUNK * EMB)], sem_s
        ).wait()

    return k(x_flat, table_flat)


def kernel(x, table):
    b, h = x.shape
    n_rows = table.shape[0]
    out = _sc_embedding_lookup(
        x.reshape(b * h), table.reshape(n_rows * EMB), n_rows
    )
    return out.reshape(b, h, EMB)


# speedup vs baseline: 2.4001x; 2.4001x over previous
import functools

import jax
import jax.numpy as jnp
from jax import lax
from jax.experimental import pallas as pl
from jax.experimental.pallas import tpu as pltpu
from jax.experimental.pallas import tpu_sc as plsc

EMB = 128
CHUNK = 400
IB = 16
LANES = 16


def _sc_embedding_lookup(x_flat, table_flat, n_rows):
    n = x_flat.shape[0]
    info = plsc.get_sparse_core_info()
    nw = info.num_cores * info.num_subcores
    per_w = n // nw
    n_iters = per_w // CHUNK
    assert per_w % CHUNK == 0 and n % nw == 0 and n_iters % IB == 0
    assert n_iters % 2 == 0

    mesh = plsc.VectorSubcoreMesh(core_axis_name="c", subcore_axis_name="s")

    @functools.partial(
        pl.kernel,
        mesh=mesh,
        compiler_params=pltpu.CompilerParams(needs_layout_passes=False),
        out_type=jax.ShapeDtypeStruct((n * EMB,), jnp.float32),
        scratch_types=[
            pltpu.VMEM((n_rows * EMB,), jnp.float32),
            pltpu.VMEM((IB * CHUNK,), jnp.int32),
            pltpu.VMEM((CHUNK * EMB,), jnp.float32),
            pltpu.VMEM((CHUNK * EMB,), jnp.float32),
            pltpu.SemaphoreType.DMA,
        ],
    )
    def k(x_hbm, table_hbm, out_hbm, table_v, idx_v, rows0, rows1, sem_s):
        wid = lax.axis_index("s") * info.num_cores + lax.axis_index("c")
        base = wid * per_w
        pltpu.sync_copy(table_hbm, table_v)
        bufs = (rows0, rows1)

        def expand_chunk(i, buf):
            j = i % IB

            @plsc.parallel_loop(0, CHUNK // LANES)
            def g_body(g):
                idx16 = idx_v[pl.ds(j * CHUNK + g * LANES, LANES)]
                base_l = idx16 << 7
                base_s = (lax.iota(jnp.int32, LANES) + g * LANES) << 7
                for c in range(EMB):
                    v = plsc.load_gather(table_v, [base_l + c])
                    plsc.store_scatter(buf, [base_s + c], v)

        def body(i2, carry):
            for s in range(2):
                i = i2 * 2 + s
                buf = bufs[s]
                start = base + i * CHUNK

                @pl.when(i % IB == 0)
                def _():
                    pltpu.sync_copy(x_hbm.at[pl.ds(start, IB * CHUNK)], idx_v)

                expand_chunk(i, buf)

                @pl.when(i >= 1)
                def _():
                    pltpu.make_async_copy(
                        bufs[1 - s],
                        out_hbm.at[pl.ds(base * EMB, CHUNK * EMB)],
                        sem_s,
                    ).wait()

                pltpu.async_copy(
                    buf, out_hbm.at[pl.ds(start * EMB, CHUNK * EMB)], sem_s
                )
            return carry

        lax.fori_loop(0, n_iters // 2, body, 0)
        pltpu.make_async_copy(
            bufs[1], out_hbm.at[pl.ds(base * EMB, CHUNK * EMB)], sem_s
        ).wait()

    return k(x_flat, table_flat)


def kernel(x, table):
    b, h = x.shape
    n_rows = table.shape[0]
    out = _sc_embedding_lookup(
        x.reshape(b * h), table.reshape(n_rows * EMB), n_rows
    )
    return out.reshape(b, h, EMB)


# speedup vs baseline: 21.1721x; 8.8214x over previous
import functools

import jax
import jax.numpy as jnp
from jax import lax
from jax.experimental import pallas as pl
from jax.experimental.pallas import tpu as pltpu
from jax.experimental.pallas import tpu_sc as plsc

EMB = 128
CHUNK = 400
IB = 16


def _sc_embedding_lookup(x_flat, table):
    n = x_flat.shape[0]
    n_rows = table.shape[0]
    info = plsc.get_sparse_core_info()
    nw = info.num_cores * info.num_subcores
    per_w = n // nw
    n_iters = per_w // CHUNK
    assert per_w % CHUNK == 0 and n % nw == 0 and n_iters % IB == 0
    assert n_iters % 2 == 0

    mesh = plsc.VectorSubcoreMesh(core_axis_name="c", subcore_axis_name="s")

    @functools.partial(
        pl.kernel,
        mesh=mesh,
        compiler_params=pltpu.CompilerParams(needs_layout_passes=False),
        out_type=jax.ShapeDtypeStruct((n, EMB), jnp.float32),
        scratch_types=[
            pltpu.VMEM_SHARED((n_rows, EMB), jnp.float32),
            pltpu.VMEM((IB * CHUNK,), jnp.int32),
            pltpu.VMEM((CHUNK, EMB), jnp.float32),
            pltpu.VMEM((CHUNK, EMB), jnp.float32),
            pltpu.SemaphoreType.DMA,
            pltpu.SemaphoreType.DMA,
        ],
    )
    def k(x_hbm, table_hbm, out_hbm, table_v, idx_v, rows0, rows1, sem_g, sem_s):
        wid = lax.axis_index("s") * info.num_cores + lax.axis_index("c")
        base = wid * per_w

        @pl.when(lax.axis_index("s") == 0)
        def _():
            pltpu.sync_copy(table_hbm, table_v)

        plsc.subcore_barrier()
        bufs = (rows0, rows1)

        def body(i2, carry):
            for s in range(2):
                i = i2 * 2 + s
                buf = bufs[s]
                start = base + i * CHUNK
                j = i % IB

                @pl.when(j == 0)
                def _():
                    pltpu.sync_copy(x_hbm.at[pl.ds(start, IB * CHUNK)], idx_v)

                pltpu.async_copy(
                    table_v.at[idx_v.at[pl.ds(j * CHUNK, CHUNK)]],
                    buf,
                    sem_g,
                ).wait()

                @pl.when(i >= 1)
                def _():
                    pltpu.make_async_copy(
                        bufs[1 - s],
                        out_hbm.at[pl.ds(base, CHUNK)],
                        sem_s,
                    ).wait()

                pltpu.async_copy(
                    buf, out_hbm.at[pl.ds(start, CHUNK)], sem_s
                )
            return carry

        lax.fori_loop(0, n_iters // 2, body, 0)
        pltpu.make_async_copy(
            bufs[1], out_hbm.at[pl.ds(base, CHUNK)], sem_s
        ).wait()

    return k(x_flat, table)


def kernel(x, table):
    b, h = x.shape
    out = _sc_embedding_lookup(x.reshape(b * h), table)
    return out.reshape(b, h, EMB)


# speedup vs baseline: 21.3410x; 1.0080x over previous
import functools

import jax
import jax.numpy as jnp
from jax import lax
from jax.experimental import pallas as pl
from jax.experimental.pallas import tpu as pltpu
from jax.experimental.pallas import tpu_sc as plsc

EMB = 128
CHUNK = 400
IB = 16


def _sc_embedding_lookup(x_flat, table):
    n = x_flat.shape[0]
    n_rows = table.shape[0]
    info = plsc.get_sparse_core_info()
    nw = info.num_cores * info.num_subcores
    per_w = n // nw
    n_iters = per_w // CHUNK
    n_blocks = n_iters // IB
    assert per_w % CHUNK == 0 and n % nw == 0 and n_iters % IB == 0
    assert IB % 2 == 0 and n_blocks % 2 == 0

    mesh = plsc.VectorSubcoreMesh(core_axis_name="c", subcore_axis_name="s")

    @functools.partial(
        pl.kernel,
        mesh=mesh,
        compiler_params=pltpu.CompilerParams(needs_layout_passes=False),
        out_type=jax.ShapeDtypeStruct((n, EMB), jnp.float32),
        scratch_types=[
            pltpu.VMEM_SHARED((n_rows, EMB), jnp.float32),
            pltpu.VMEM((IB * CHUNK,), jnp.int32),
            pltpu.VMEM((IB * CHUNK,), jnp.int32),
            pltpu.VMEM((CHUNK, EMB), jnp.float32),
            pltpu.VMEM((CHUNK, EMB), jnp.float32),
            pltpu.SemaphoreType.DMA,
            pltpu.SemaphoreType.DMA,
            pltpu.SemaphoreType.DMA,
            pltpu.SemaphoreType.DMA,
        ],
    )
    def k(x_hbm, table_hbm, out_hbm, table_v, idx0, idx1, rows0, rows1,
          sem_i0, sem_i1, sem_g, sem_s):
        wid = lax.axis_index("s") * info.num_cores + lax.axis_index("c")
        base = wid * per_w

        @pl.when(lax.axis_index("s") == 0)
        def _():
            pltpu.sync_copy(table_hbm, table_v)

        plsc.subcore_barrier()

        rbufs = (rows0, rows1)
        ibufs = (idx0, idx1)
        isems = (sem_i0, sem_i1)

        pltpu.async_copy(x_hbm.at[pl.ds(base, IB * CHUNK)], idx0, sem_i0)

        def do_block(p, parity, ib, isem):
            b = p * 2 + parity
            blk_start = base + b * IB * CHUNK

            @pl.when(b + 1 < n_blocks)
            def _():
                pltpu.async_copy(
                    x_hbm.at[pl.ds(blk_start + IB * CHUNK, IB * CHUNK)],
                    ibufs[1 - parity],
                    isems[1 - parity],
                )

            pltpu.make_async_copy(
                x_hbm.at[pl.ds(blk_start, IB * CHUNK)], ib, isem
            ).wait()

            for c in range(IB):
                buf = rbufs[c % 2]
                start = blk_start + c * CHUNK

                pltpu.async_copy(
                    table_v.at[ib.at[pl.ds(c * CHUNK, CHUNK)]], buf, sem_g
                ).wait()

                def drain_prev():
                    pltpu.make_async_copy(
                        rbufs[1 - (c % 2)],
                        out_hbm.at[pl.ds(base, CHUNK)],
                        sem_s,
                    ).wait()

                if c == 0 and parity == 0:
                    pl.when(p > 0)(drain_prev)
                else:
                    drain_prev()

                pltpu.async_copy(buf, out_hbm.at[pl.ds(start, CHUNK)], sem_s)

        def body(p, carry):
            do_block(p, 0, idx0, sem_i0)
            do_block(p, 1, idx1, sem_i1)
            return carry

        lax.fori_loop(0, n_blocks // 2, body, 0)
        pltpu.make_async_copy(
            rbufs[(n_iters - 1) % 2], out_hbm.at[pl.ds(base, CHUNK)], sem_s
        ).wait()

    return k(x_flat, table)


def kernel(x, table):
    b, h = x.shape
    out = _sc_embedding_lookup(x.reshape(b * h), table)
    return out.reshape(b, h, EMB)


# speedup vs baseline: 21.3499x; 1.0004x over previous
import functools

import jax
import jax.numpy as jnp
from jax import lax
from jax.experimental import pallas as pl
from jax.experimental.pallas import tpu as pltpu
from jax.experimental.pallas import tpu_sc as plsc

EMB = 128
CHUNK = 400
IB = 16


def _sc_embedding_lookup(x_flat, table):
    n = x_flat.shape[0]
    n_rows = table.shape[0]
    info = plsc.get_sparse_core_info()
    nw = info.num_cores * info.num_subcores
    per_w = n // nw
    n_iters = per_w // CHUNK
    n_blocks = n_iters // IB
    assert per_w % CHUNK == 0 and n % nw == 0 and n_iters % IB == 0
    assert IB % 2 == 0 and IB >= 4 and n_blocks % 2 == 0

    mesh = plsc.VectorSubcoreMesh(core_axis_name="c", subcore_axis_name="s")

    @functools.partial(
        pl.kernel,
        mesh=mesh,
        compiler_params=pltpu.CompilerParams(needs_layout_passes=False),
        out_type=jax.ShapeDtypeStruct((n, EMB), jnp.float32),
        scratch_types=[
            pltpu.VMEM_SHARED((n_rows, EMB), jnp.float32),
            pltpu.VMEM((IB * CHUNK,), jnp.int32),
            pltpu.VMEM((IB * CHUNK,), jnp.int32),
            pltpu.VMEM((CHUNK, EMB), jnp.float32),
            pltpu.VMEM((CHUNK, EMB), jnp.float32),
            pltpu.SemaphoreType.DMA,
            pltpu.SemaphoreType.DMA,
            pltpu.SemaphoreType.DMA,
            pltpu.SemaphoreType.DMA,
            pltpu.SemaphoreType.DMA,
        ],
    )
    def k(x_hbm, table_hbm, out_hbm, table_v, idx0, idx1, rows0, rows1,
          sem_i0, sem_i1, sem_g0, sem_g1, sem_s):
        wid = lax.axis_index("s") * info.num_cores + lax.axis_index("c")
        base = wid * per_w

        @pl.when(lax.axis_index("s") == 0)
        def _():
            pltpu.sync_copy(table_hbm, table_v)

        plsc.subcore_barrier()

        rbufs = (rows0, rows1)
        ibufs = (idx0, idx1)
        isems = (sem_i0, sem_i1)
        gsems = (sem_g0, sem_g1)

        def gather(ib, c, buf, gsem):
            pltpu.async_copy(
                table_v.at[ib.at[pl.ds(c * CHUNK, CHUNK)]], buf, gsem
            )

        def wait_store():
            pltpu.make_async_copy(
                rows0, out_hbm.at[pl.ds(base, CHUNK)], sem_s
            ).wait()

        pltpu.async_copy(x_hbm.at[pl.ds(base, IB * CHUNK)], idx0, sem_i0)

        def do_block(p, parity, ib, isem):
            b = p * 2 + parity
            i0 = b * IB
            blk_start = base + i0 * CHUNK

            @pl.when(b + 1 < n_blocks)
            def _():
                pltpu.async_copy(
                    x_hbm.at[pl.ds(blk_start + IB * CHUNK, IB * CHUNK)],
                    ibufs[1 - parity],
                    isems[1 - parity],
                )

            pltpu.make_async_copy(
                x_hbm.at[pl.ds(blk_start, IB * CHUNK)], ib, isem
            ).wait()

            def head():
                wait_store()

            if parity == 0:
                pl.when(p > 0)(head)
            else:
                head()
            gather(ib, 0, rbufs[0], gsems[0])
            if parity == 0:
                pl.when(p > 0)(head)
            else:
                head()
            gather(ib, 1, rbufs[1], gsems[1])

            for c in range(IB):
                i = i0 + c
                buf = rbufs[c % 2]
                pltpu.make_async_copy(
                    table_v.at[ib.at[pl.ds(c * CHUNK, CHUNK)]],
                    buf,
                    gsems[c % 2],
                ).wait()
                pltpu.async_copy(
                    buf, out_hbm.at[pl.ds(blk_start + c * CHUNK, CHUNK)], sem_s
                )
                if c + 2 < IB:
                    wait_store()
                    gather(ib, c + 2, rbufs[c % 2], gsems[c % 2])

        def body(p, carry):
            do_block(p, 0, idx0, sem_i0)
            do_block(p, 1, idx1, sem_i1)
            return carry

        lax.fori_loop(0, n_blocks // 2, body, 0)
        wait_store()
        wait_store()

    return k(x_flat, table)


def kernel(x, table):
    b, h = x.shape
    out = _sc_embedding_lookup(x.reshape(b * h), table)
    return out.reshape(b, h, EMB)
